# Initial kernel scaffold; baseline (speedup 1.0000x reference)
import jax
import jax.numpy as jnp
from jax.experimental import pallas as pl


def kernel(x_nchw, pslab):
    raise NotImplementedError("write your pallas kernel here")


# speedup vs baseline: 1.1411x; 1.1411x over previous
import functools

import jax
import jax.numpy as jnp
from jax.experimental import pallas as pl
from jax.experimental.pallas import tpu as pltpu

_C = 128
_IMGS = 8


def _silu(v):
    return v * (1.0 / (1.0 + jnp.exp(-v)))


def _shift_pair(v, w):
    c, hw = v.shape
    z = jnp.zeros((c, w), v.dtype)
    up = jnp.concatenate([z, v[:, :hw - w]], axis=1)
    dn = jnp.concatenate([v[:, w:], z], axis=1)
    return up, dn


def _pipe_kernel(x_ref, w1_ref, w2_ref, misc_ref, w4_ref, w5_ref, o_ref, *,
                 imgs, w):
    w1 = w1_ref[...]
    w2 = w2_ref[...]
    w4 = w4_ref[...]
    w5 = w5_ref[...]
    m = misc_ref[...]
    b1 = m[:, 0:1]
    b2 = m[:, 1:2]
    b3 = m[:, 2:3]
    b4 = m[:, 3:4]
    b5 = m[:, 4:5]
    w3a = m[:, 5:6]
    w3b = m[:, 6:7]
    w3c = m[:, 7:8]

    for i in range(imgs):
        x = x_ref[i]
        x16 = x.astype(jnp.bfloat16)

        up, dn = _shift_pair(x16, w)
        xcat = jnp.concatenate([up, x16, dn], axis=0)
        y = jnp.dot(w1, xcat, preferred_element_type=jnp.float32) + b1
        y = _silu(y)

        y16 = y.astype(jnp.bfloat16)
        y = jnp.dot(w2, y16, preferred_element_type=jnp.float32) + b2
        y = _silu(y)

        upf, dnf = _shift_pair(y, w)
        y = upf * w3a + y * w3b + dnf * w3c + b3
        y = _silu(y)

        y16 = y.astype(jnp.bfloat16)
        y = jnp.dot(w4, y16, preferred_element_type=jnp.float32) + b4
        y = _silu(y)

        y16 = y.astype(jnp.bfloat16)
        up, dn = _shift_pair(y16, w)
        ycat = jnp.concatenate([up, y16, dn], axis=0)
        y = jnp.dot(w5, ycat, preferred_element_type=jnp.float32) + b5
        y = _silu(y)

        o_ref[i] = y + x


def kernel(x_nchw, pslab):
    N, C, H, W = x_nchw.shape
    HW = H * W
    xk = x_nchw.reshape(N, C, HW)

    w1 = pslab[:, 0:384].astype(jnp.bfloat16)
    w2 = pslab[:, 385:513].astype(jnp.bfloat16)
    w4 = pslab[:, 518:646].astype(jnp.bfloat16)
    w5 = pslab[:, 647:1031].astype(jnp.bfloat16)
    misc = jnp.concatenate(
        [pslab[:, 384:385], pslab[:, 513:514], pslab[:, 517:518],
         pslab[:, 646:647], pslab[:, 1031:1032], pslab[:, 514:517]], axis=1)

    out = pl.pallas_call(
        functools.partial(_pipe_kernel, imgs=_IMGS, w=W),
        out_shape=jax.ShapeDtypeStruct((N, C, HW), x_nchw.dtype),
        grid=(N // _IMGS,),
        in_specs=[
            pl.BlockSpec((_IMGS, C, HW), lambda i: (i, 0, 0)),
            pl.BlockSpec((C, 3 * C), lambda i: (0, 0)),
            pl.BlockSpec((C, C), lambda i: (0, 0)),
            pl.BlockSpec((C, 8), lambda i: (0, 0)),
            pl.BlockSpec((C, C), lambda i: (0, 0)),
            pl.BlockSpec((C, 3 * C), lambda i: (0, 0)),
        ],
        out_specs=pl.BlockSpec((_IMGS, C, HW), lambda i: (i, 0, 0)),
        compiler_params=pltpu.CompilerParams(
            dimension_semantics=("parallel",),
            vmem_limit_bytes=64 * 1024 * 1024),
    )(xk, w1, w2, misc, w4, w5)

    return out.reshape(N, C, H, W)


# speedup vs baseline: 1.2787x; 1.1205x over previous
import functools

import jax
import jax.numpy as jnp
from jax.experimental import pallas as pl
from jax.experimental.pallas import tpu as pltpu

_C = 128
_IMGS = 8


def _silu_of_2h(h):
    return h + h * jnp.tanh(h)


def _shift_pair(v, w):
    c, hw = v.shape
    z = jnp.zeros((c, w), v.dtype)
    up = jnp.concatenate([z, v[:, :hw - w]], axis=1)
    dn = jnp.concatenate([v[:, w:], z], axis=1)
    return up, dn


def _pipe_kernel(x_ref, w1_ref, w2_ref, misc_ref, w4_ref, w5_ref, o_ref, *,
                 imgs, w):
    w1 = w1_ref[...]
    w2 = w2_ref[...]
    w4 = w4_ref[...]
    w5 = w5_ref[...]
    m = misc_ref[...]
    b2 = m[:, 0:1]
    b3 = m[:, 1:2]
    b4 = m[:, 2:3]
    w3a = m[:, 3:4]
    w3b = m[:, 4:5]
    w3c = m[:, 5:6]
    hw = o_ref.shape[-1]
    ones = jnp.ones((1, hw), jnp.bfloat16)

    for i in range(imgs):
        x = x_ref[i]
        x16 = x.astype(jnp.bfloat16)

        up, dn = _shift_pair(x16, w)
        xcat = jnp.concatenate([up, x16, dn, ones], axis=0)
        h = jnp.dot(w1, xcat, preferred_element_type=jnp.float32)
        y = _silu_of_2h(h)

        y16 = y.astype(jnp.bfloat16)
        h = jnp.dot(w2, y16, preferred_element_type=jnp.float32) + b2
        y = _silu_of_2h(h)

        upf, dnf = _shift_pair(y, w)
        h = upf * w3a + y * w3b + dnf * w3c + b3
        y = _silu_of_2h(h)

        y16 = y.astype(jnp.bfloat16)
        h = jnp.dot(w4, y16, preferred_element_type=jnp.float32) + b4
        y = _silu_of_2h(h)

        y16 = y.astype(jnp.bfloat16)
        up, dn = _shift_pair(y16, w)
        ycat = jnp.concatenate([up, y16, dn, ones], axis=0)
        h = jnp.dot(w5, ycat, preferred_element_type=jnp.float32)
        y = _silu_of_2h(h)

        o_ref[i] = y + x


def kernel(x_nchw, pslab):
    N, C, H, W = x_nchw.shape
    HW = H * W
    xk = x_nchw.reshape(N, C, HW)

    half = pslab * 0.5
    w1 = half[:, 0:385].astype(jnp.bfloat16)
    w2 = half[:, 385:513].astype(jnp.bfloat16)
    w4 = half[:, 518:646].astype(jnp.bfloat16)
    w5 = half[:, 647:1032].astype(jnp.bfloat16)
    misc = jnp.concatenate(
        [half[:, 513:514], half[:, 517:518], half[:, 646:647],
         half[:, 514:517], half[:, 513:515]], axis=1)

    out = pl.pallas_call(
        functools.partial(_pipe_kernel, imgs=_IMGS, w=W),
        out_shape=jax.ShapeDtypeStruct((N, C, HW), x_nchw.dtype),
        grid=(N // _IMGS,),
        in_specs=[
            pl.BlockSpec((_IMGS, C, HW), lambda i: (i, 0, 0)),
            pl.BlockSpec((C, 3 * C + 1), lambda i: (0, 0)),
            pl.BlockSpec((C, C), lambda i: (0, 0)),
            pl.BlockSpec((C, 8), lambda i: (0, 0)),
            pl.BlockSpec((C, C), lambda i: (0, 0)),
            pl.BlockSpec((C, 3 * C + 1), lambda i: (0, 0)),
        ],
        out_specs=pl.BlockSpec((_IMGS, C, HW), lambda i: (i, 0, 0)),
        compiler_params=pltpu.CompilerParams(
            dimension_semantics=("parallel",),
            vmem_limit_bytes=64 * 1024 * 1024),
    )(xk, w1, w2, misc, w4, w5)

    return out.reshape(N, C, H, W)
